# baseline (speedup 1.0000x reference)
import jax
import jax.numpy as jnp
from jax import lax
from jax.experimental import pallas as pl
from jax.experimental.pallas import tpu as pltpu
from jax.experimental.pallas import tpu_sc as plsc

_EPS = 1e-8
_NSEG = 8
_D = 256
_LANES = 16
_CHUNK = 128
_B = 16
_T = 4096
_TC_TOK = 3072
_NTILES = 32
_TOK_PER_TILE = (_T - _TC_TOK) // 2


def _sc_body(attr_hbm, feats_hbm, out_hbm, attr_v, x_v, acc_v, sem0, sem1):
    nsteps = _TOK_PER_TILE // _CHUNK
    nj = _D // _LANES
    sid = lax.axis_index("s")
    cid = lax.axis_index("c")
    wid = sid * 2 + cid
    base = sid * _T + _TC_TOK + cid * _TOK_PER_TILE

    zeros = jnp.zeros((_LANES,), jnp.float32)

    def zbody(t, _):
        acc_v[t >> 4, pl.ds((t & 15) * _LANES, _LANES)] = zeros
        return 0

    lax.fori_loop(0, _NSEG * nj, zbody, 0)

    pltpu.sync_copy(attr_hbm.at[pl.ds(base, _TOK_PER_TILE)], attr_v)

    sems = (sem0, sem1)

    def issue(chunk, b):
        return pltpu.async_copy(
            feats_hbm.at[pl.ds(base + chunk * _CHUNK, _CHUNK)],
            x_v.at[b], sems[b])

    def consume(chunk, b):
        pltpu.make_async_copy(
            feats_hbm.at[pl.ds(base + chunk * _CHUNK, _CHUNK)],
            x_v.at[b], sems[b]).wait()

        def load_row(t):
            return [x_v[b, t, pl.ds(j * _LANES, _LANES)] for j in range(nj)]

        def grp_body(gi, _):
            av = attr_v[pl.ds(chunk * _CHUNK + gi * _LANES, _LANES)]
            t0 = gi * _LANES
            row = load_row(t0)
            for k in range(_LANES):
                a = av[k]
                nxt = load_row(t0 + k + 1) if k + 1 < _LANES else None
                for j in range(nj):
                    plsc.addupdate(
                        acc_v.at[a, pl.ds(j * _LANES, _LANES)], row[j])
                row = nxt
            return 0

        lax.fori_loop(0, _CHUNK // _LANES, grp_body, 0)

    issue(0, 0)
    issue(1, 1)

    @pl.loop(0, nsteps, step=2)
    def _(g):
        for b in range(2):
            consume(g + b, b)
            issue(jnp.minimum(g + b + 2, nsteps - 1), b)

    for b in range(2):
        pltpu.make_async_copy(
            feats_hbm.at[pl.ds(base, _CHUNK)], x_v.at[b], sems[b]).wait()

    pltpu.sync_copy(acc_v, out_hbm.at[wid])


def _sc_segment_sums(flat_attr, flat_feats):
    run = pl.kernel(
        _sc_body,
        out_type=jax.ShapeDtypeStruct((_NTILES, _NSEG, _D), jnp.float32),
        mesh=plsc.VectorSubcoreMesh(core_axis_name="c", subcore_axis_name="s"),
        scratch_types=[
            pltpu.VMEM((_TOK_PER_TILE,), jnp.int32),
            pltpu.VMEM((2, _CHUNK, _D), jnp.float32),
            pltpu.VMEM((_NSEG, _D), jnp.float32),
            pltpu.SemaphoreType.DMA,
            pltpu.SemaphoreType.DMA,
        ],
    )
    return run(flat_attr, flat_feats)


def _tc_body(attr_ref, x_ref, out_ref):
    attr = attr_ref[0, 0, :]
    x = x_ref[0]
    seg_ids = lax.broadcasted_iota(jnp.int32, (_NSEG, _TC_TOK), 0)
    mask = (seg_ids == attr[None, :]).astype(jnp.float32)
    out_ref[0] = jnp.dot(mask, x, preferred_element_type=jnp.float32)


def _tc_segment_sums(attr3, text_feats):
    return pl.pallas_call(
        _tc_body,
        grid=(_B,),
        in_specs=[
            pl.BlockSpec((1, 1, _TC_TOK), lambda b: (b, 0, 0)),
            pl.BlockSpec((1, _TC_TOK, _D), lambda b: (b, 0, 0)),
        ],
        out_specs=pl.BlockSpec((1, _NSEG, _D), lambda b: (b, 0, 0)),
        out_shape=jax.ShapeDtypeStruct((_B, _NSEG, _D), jnp.float32),
    )(attr3, text_feats)


def _epilogue_body(attr_ref, tc_ref, sc_ref, vg_ref, out_ref):
    attr = attr_ref[...]
    tc = tc_ref[...]
    sc = sc_ref[...]
    vgs = vg_ref[...]

    seg_sums = tc + sc[:, 0] + sc[:, 1]

    cols = []
    for i in range(_NSEG):
        cols.append(jnp.sum((attr == i).astype(jnp.float32), axis=1,
                            keepdims=True))
    counts = jnp.concatenate(cols, axis=1)

    mean = seg_sums / counts[:, :, None]
    num = jnp.sum(mean * vgs[:, None, :], axis=2)
    norm_m = jnp.sqrt(jnp.sum(mean * mean, axis=2))
    norm_vg = jnp.sqrt(jnp.sum(vgs * vgs, axis=1, keepdims=True))
    denom = jnp.maximum(norm_vg, _EPS) * jnp.maximum(norm_m, _EPS)
    cos = num / denom

    ids = lax.broadcasted_iota(jnp.int32, (_B, _NSEG), 1)
    present = counts > 0.0
    max_attr = jnp.max(jnp.where(present, ids, 0), axis=1, keepdims=True)
    valid = (ids >= 1) & (ids <= max_attr)
    cs = (jnp.sum(jnp.where(valid, cos, 0.0), axis=1, keepdims=True)
          / max_attr.astype(jnp.float32))
    has_any = max_attr > 0
    loss_b = jnp.where(has_any, 1.0 - cs, 0.0)
    total = jnp.sum(loss_b)
    cnt = jnp.sum(has_any.astype(jnp.float32))
    out_ref[0, 0] = total / cnt


def kernel(attributes, text_feats, Vgs):
    B, T = attributes.shape
    attr = attributes.astype(jnp.int32)
    sc_part = _sc_segment_sums(attr.reshape(B * T),
                               text_feats.reshape(B * T, _D))
    tc_part = _tc_segment_sums(attr.reshape(B, 1, T), text_feats)
    out = pl.pallas_call(
        _epilogue_body,
        in_specs=[
            pl.BlockSpec(memory_space=pltpu.VMEM),
            pl.BlockSpec(memory_space=pltpu.VMEM),
            pl.BlockSpec(memory_space=pltpu.VMEM),
            pl.BlockSpec(memory_space=pltpu.VMEM),
        ],
        out_specs=pl.BlockSpec(memory_space=pltpu.SMEM),
        out_shape=jax.ShapeDtypeStruct((1, 1), jnp.float32),
    )(attr, tc_part, sc_part.reshape(B, 2, _NSEG, _D), Vgs)
    return out[0, 0]


# speedup vs baseline: 1.0445x; 1.0445x over previous
import jax
import jax.numpy as jnp
from jax import lax
from jax.experimental import pallas as pl
from jax.experimental.pallas import tpu as pltpu
from jax.experimental.pallas import tpu_sc as plsc

_EPS = 1e-8
_NSEG = 8
_D = 256
_LANES = 16
_CHUNK = 128
_B = 16
_T = 4096
_TC_TOK = 3072
_NTILES = 32
_TOK_PER_TILE = (_T - _TC_TOK) // 2


def _sc_body(attr_hbm, feats_hbm, out_hbm, attr_v, x_v, acc_v, sem0, sem1):
    nsteps = _TOK_PER_TILE // _CHUNK
    nj = _D // _LANES
    sid = lax.axis_index("s")
    cid = lax.axis_index("c")
    wid = sid * 2 + cid
    base = sid * _T + _TC_TOK + cid * _TOK_PER_TILE

    zeros = jnp.zeros((_LANES,), jnp.float32)
    for i in range(_NSEG):
        for j in range(nj):
            acc_v[i, pl.ds(j * _LANES, _LANES)] = zeros

    pltpu.sync_copy(attr_hbm.at[pl.ds(base, _TOK_PER_TILE)], attr_v)

    sems = (sem0, sem1)

    def issue(chunk, b):
        return pltpu.async_copy(
            feats_hbm.at[pl.ds(base + chunk * _CHUNK, _CHUNK)],
            x_v.at[b], sems[b])

    def consume(chunk, b):
        pltpu.make_async_copy(
            feats_hbm.at[pl.ds(base + chunk * _CHUNK, _CHUNK)],
            x_v.at[b], sems[b]).wait()

        def load_row(t):
            return [x_v[b, t, pl.ds(j * _LANES, _LANES)] for j in range(nj)]

        def grp_body(gi, _):
            av = attr_v[pl.ds(chunk * _CHUNK + gi * _LANES, _LANES)]
            t0 = gi * _LANES
            row = load_row(t0)
            for k in range(_LANES):
                a = av[k]
                nxt = load_row(t0 + k + 1) if k + 1 < _LANES else None
                for j in range(nj):
                    plsc.addupdate(
                        acc_v.at[a, pl.ds(j * _LANES, _LANES)], row[j])
                row = nxt
            return 0

        lax.fori_loop(0, _CHUNK // _LANES, grp_body, 0)

    issue(0, 0)
    issue(1, 1)

    @pl.loop(0, nsteps - 2, step=2)
    def _(g):
        for b in range(2):
            consume(g + b, b)
            issue(g + b + 2, b)

    for b in range(2):
        consume(nsteps - 2 + b, b)

    pltpu.sync_copy(acc_v, out_hbm.at[wid])


def _sc_segment_sums(flat_attr, flat_feats):
    run = pl.kernel(
        _sc_body,
        out_type=jax.ShapeDtypeStruct((_NTILES, _NSEG, _D), jnp.float32),
        mesh=plsc.VectorSubcoreMesh(core_axis_name="c", subcore_axis_name="s"),
        scratch_types=[
            pltpu.VMEM((_TOK_PER_TILE,), jnp.int32),
            pltpu.VMEM((2, _CHUNK, _D), jnp.float32),
            pltpu.VMEM((_NSEG, _D), jnp.float32),
            pltpu.SemaphoreType.DMA,
            pltpu.SemaphoreType.DMA,
        ],
    )
    return run(flat_attr, flat_feats)


def _tc_body(attr_ref, x_ref, out_ref):
    attr = attr_ref[: _TC_TOK]
    x = x_ref[0]
    seg_ids = lax.broadcasted_iota(jnp.int32, (_NSEG, _TC_TOK), 0)
    mask = (seg_ids == attr[None, :]).astype(jnp.float32)
    out_ref[0] = jnp.dot(mask, x, preferred_element_type=jnp.float32)


def _tc_segment_sums(flat_attr, text_feats):
    return pl.pallas_call(
        _tc_body,
        grid=(_B,),
        in_specs=[
            pl.BlockSpec((_T,), lambda b: (b,)),
            pl.BlockSpec((1, _TC_TOK, _D), lambda b: (b, 0, 0)),
        ],
        out_specs=pl.BlockSpec((1, _NSEG, _D), lambda b: (b, 0, 0)),
        out_shape=jax.ShapeDtypeStruct((_B, _NSEG, _D), jnp.float32),
    )(flat_attr, text_feats)


def _epilogue_body(attr_ref, tc_ref, sc_ref, vg_ref, out_ref):
    attr = attr_ref[...].reshape(_B, _T)
    tc = tc_ref[...]
    sc = sc_ref[...]
    vgs = vg_ref[...]

    seg_sums = tc + sc[:, 0] + sc[:, 1]

    cols = []
    for i in range(_NSEG):
        cols.append(jnp.sum((attr == i).astype(jnp.float32), axis=1,
                            keepdims=True))
    counts = jnp.concatenate(cols, axis=1)

    mean = seg_sums / counts[:, :, None]
    num = jnp.sum(mean * vgs[:, None, :], axis=2)
    norm_m = jnp.sqrt(jnp.sum(mean * mean, axis=2))
    norm_vg = jnp.sqrt(jnp.sum(vgs * vgs, axis=1, keepdims=True))
    denom = jnp.maximum(norm_vg, _EPS) * jnp.maximum(norm_m, _EPS)
    cos = num / denom

    ids = lax.broadcasted_iota(jnp.int32, (_B, _NSEG), 1)
    present = counts > 0.0
    max_attr = jnp.max(jnp.where(present, ids, 0), axis=1, keepdims=True)
    valid = (ids >= 1) & (ids <= max_attr)
    cs = (jnp.sum(jnp.where(valid, cos, 0.0), axis=1, keepdims=True)
          / max_attr.astype(jnp.float32))
    has_any = max_attr > 0
    loss_b = jnp.where(has_any, 1.0 - cs, 0.0)
    total = jnp.sum(loss_b)
    cnt = jnp.sum(has_any.astype(jnp.float32))
    out_ref[0, 0] = total / cnt


def kernel(attributes, text_feats, Vgs):
    B, T = attributes.shape
    attr = attributes.astype(jnp.int32).reshape(B * T)
    sc_part = _sc_segment_sums(attr, text_feats.reshape(B * T, _D))
    tc_part = _tc_segment_sums(attr, text_feats)
    out = pl.pallas_call(
        _epilogue_body,
        in_specs=[
            pl.BlockSpec(memory_space=pltpu.VMEM),
            pl.BlockSpec(memory_space=pltpu.VMEM),
            pl.BlockSpec(memory_space=pltpu.VMEM),
            pl.BlockSpec(memory_space=pltpu.VMEM),
        ],
        out_specs=pl.BlockSpec(memory_space=pltpu.SMEM),
        out_shape=jax.ShapeDtypeStruct((1, 1), jnp.float32),
    )(attr, tc_part, sc_part.reshape(B, 2, _NSEG, _D), Vgs)
    return out[0, 0]


# speedup vs baseline: 1.0686x; 1.0231x over previous
import jax
import jax.numpy as jnp
from jax import lax
from jax.experimental import pallas as pl
from jax.experimental.pallas import tpu as pltpu
from jax.experimental.pallas import tpu_sc as plsc

_EPS = 1e-8
_NSEG = 8
_D = 256
_LANES = 16
_CHUNK = 128
_B = 16
_T = 4096
_TC_TOK = 3072
_NTILES = 32
_TOK_PER_TILE = (_T - _TC_TOK) // 2


def _sc_body(attr_hbm, feats_hbm, out_hbm, attr_v, x_v, acc_v, sem0, sem1):
    nsteps = _TOK_PER_TILE // _CHUNK
    nj = _D // _LANES
    sid = lax.axis_index("s")
    cid = lax.axis_index("c")
    wid = sid * 2 + cid
    base = sid * _T + _TC_TOK + cid * _TOK_PER_TILE

    zeros = jnp.zeros((_LANES,), jnp.float32)
    for i in range(_NSEG):
        for j in range(nj):
            acc_v[i, pl.ds(j * _LANES, _LANES)] = zeros

    pltpu.sync_copy(
        attr_hbm.at[sid, 0, pl.ds(_TC_TOK + cid * _TOK_PER_TILE,
                                  _TOK_PER_TILE)], attr_v)

    sems = (sem0, sem1)

    def issue(chunk, b):
        return pltpu.async_copy(
            feats_hbm.at[pl.ds(base + chunk * _CHUNK, _CHUNK)],
            x_v.at[b], sems[b])

    def consume(chunk, b):
        pltpu.make_async_copy(
            feats_hbm.at[pl.ds(base + chunk * _CHUNK, _CHUNK)],
            x_v.at[b], sems[b]).wait()

        def load_row(t):
            return [x_v[b, t, pl.ds(j * _LANES, _LANES)] for j in range(nj)]

        def grp_body(gi, _):
            av = attr_v[pl.ds(chunk * _CHUNK + gi * _LANES, _LANES)]
            t0 = gi * _LANES
            row = load_row(t0)
            for k in range(_LANES):
                a = av[k]
                nxt = load_row(t0 + k + 1) if k + 1 < _LANES else None
                for j in range(nj):
                    plsc.addupdate(
                        acc_v.at[a, pl.ds(j * _LANES, _LANES)], row[j])
                row = nxt
            return 0

        lax.fori_loop(0, _CHUNK // _LANES, grp_body, 0)

    issue(0, 0)
    issue(1, 1)

    @pl.loop(0, nsteps - 2, step=2)
    def _(g):
        for b in range(2):
            consume(g + b, b)
            issue(g + b + 2, b)

    for b in range(2):
        consume(nsteps - 2 + b, b)

    pltpu.sync_copy(acc_v, out_hbm.at[wid])


def _sc_segment_sums(attr3, flat_feats):
    run = pl.kernel(
        _sc_body,
        out_type=jax.ShapeDtypeStruct((_NTILES, _NSEG, _D), jnp.float32),
        mesh=plsc.VectorSubcoreMesh(core_axis_name="c", subcore_axis_name="s"),
        scratch_types=[
            pltpu.VMEM((_TOK_PER_TILE,), jnp.int32),
            pltpu.VMEM((2, _CHUNK, _D), jnp.float32),
            pltpu.VMEM((_NSEG, _D), jnp.float32),
            pltpu.SemaphoreType.DMA,
            pltpu.SemaphoreType.DMA,
        ],
    )
    return run(attr3, flat_feats)


def _tc_body(attr_ref, x_ref, out_ref, cnt_ref):
    attr = attr_ref[0, 0, :]
    x = x_ref[0]
    seg_ids = lax.broadcasted_iota(jnp.int32, (_NSEG, _T), 0)
    mask = (seg_ids == attr[None, :]).astype(jnp.float32)
    out_ref[0] = jnp.dot(mask[:, :_TC_TOK], x,
                         preferred_element_type=jnp.float32)
    cnt_ref[0] = jnp.broadcast_to(
        jnp.sum(mask, axis=1, keepdims=True), (_NSEG, 128))


def _tc_segment_sums(attr3, text_feats):
    return pl.pallas_call(
        _tc_body,
        grid=(_B,),
        in_specs=[
            pl.BlockSpec((1, 1, _T), lambda b: (b, 0, 0)),
            pl.BlockSpec((1, _TC_TOK, _D), lambda b: (b, 0, 0)),
        ],
        out_specs=[
            pl.BlockSpec((1, _NSEG, _D), lambda b: (b, 0, 0)),
            pl.BlockSpec((1, _NSEG, 128), lambda b: (b, 0, 0)),
        ],
        out_shape=[
            jax.ShapeDtypeStruct((_B, _NSEG, _D), jnp.float32),
            jax.ShapeDtypeStruct((_B, _NSEG, 128), jnp.float32),
        ],
    )(attr3, text_feats)


def _epilogue_body(cnt_ref, tc_ref, sc_ref, vg_ref, out_ref):
    counts = cnt_ref[:, :, 0]
    tc = tc_ref[...]
    sc = sc_ref[...]
    vgs = vg_ref[...]

    seg_sums = tc + sc[:, 0] + sc[:, 1]

    mean = seg_sums / counts[:, :, None]
    num = jnp.sum(mean * vgs[:, None, :], axis=2)
    norm_m = jnp.sqrt(jnp.sum(mean * mean, axis=2))
    norm_vg = jnp.sqrt(jnp.sum(vgs * vgs, axis=1, keepdims=True))
    denom = jnp.maximum(norm_vg, _EPS) * jnp.maximum(norm_m, _EPS)
    cos = num / denom

    ids = lax.broadcasted_iota(jnp.int32, (_B, _NSEG), 1)
    present = counts > 0.0
    max_attr = jnp.max(jnp.where(present, ids, 0), axis=1, keepdims=True)
    valid = (ids >= 1) & (ids <= max_attr)
    cs = (jnp.sum(jnp.where(valid, cos, 0.0), axis=1, keepdims=True)
          / max_attr.astype(jnp.float32))
    has_any = max_attr > 0
    loss_b = jnp.where(has_any, 1.0 - cs, 0.0)
    total = jnp.sum(loss_b)
    cnt = jnp.sum(has_any.astype(jnp.float32))
    out_ref[0, 0] = total / cnt


def kernel(attributes, text_feats, Vgs):
    B, T = attributes.shape
    attr3 = attributes.astype(jnp.int32).reshape(B, 1, T)
    sc_part = _sc_segment_sums(attr3, text_feats.reshape(B * T, _D))
    tc_sums, tc_cnt = _tc_segment_sums(attr3, text_feats)
    out = pl.pallas_call(
        _epilogue_body,
        in_specs=[
            pl.BlockSpec(memory_space=pltpu.VMEM),
            pl.BlockSpec(memory_space=pltpu.VMEM),
            pl.BlockSpec(memory_space=pltpu.VMEM),
            pl.BlockSpec(memory_space=pltpu.VMEM),
        ],
        out_specs=pl.BlockSpec(memory_space=pltpu.SMEM),
        out_shape=jax.ShapeDtypeStruct((1, 1), jnp.float32),
    )(tc_cnt, tc_sums, sc_part.reshape(B, 2, _NSEG, _D), Vgs)
    return out[0, 0]


# speedup vs baseline: 1.0730x; 1.0041x over previous
import jax
import jax.numpy as jnp
from jax import lax
from jax.experimental import pallas as pl
from jax.experimental.pallas import tpu as pltpu
from jax.experimental.pallas import tpu_sc as plsc

_EPS = 1e-8
_NSEG = 8
_D = 256
_LANES = 16
_CHUNK = 128
_B = 16
_T = 4096
_TC_TOK = 3072
_NTILES = 32
_TOK_PER_TILE = (_T - _TC_TOK) // 2


def _sc_body(attr_hbm, feats_hbm, out_hbm, attr_v, x_v, acc_v, sem0, sem1):
    nsteps = _TOK_PER_TILE // _CHUNK
    nj = _D // _LANES
    sid = lax.axis_index("s")
    cid = lax.axis_index("c")
    wid = sid * 2 + cid
    base = _TC_TOK + cid * _TOK_PER_TILE

    zeros = jnp.zeros((_LANES,), jnp.float32)
    for i in range(_NSEG):
        for j in range(nj):
            acc_v[i, pl.ds(j * _LANES, _LANES)] = zeros

    pltpu.sync_copy(
        attr_hbm.at[sid, 0, pl.ds(_TC_TOK + cid * _TOK_PER_TILE,
                                  _TOK_PER_TILE)], attr_v)

    sems = (sem0, sem1)

    def issue(chunk, b):
        return pltpu.async_copy(
            feats_hbm.at[sid, pl.ds(base + chunk * _CHUNK, _CHUNK), :],
            x_v.at[b], sems[b])

    def consume(chunk, b):
        pltpu.make_async_copy(
            feats_hbm.at[sid, pl.ds(base + chunk * _CHUNK, _CHUNK), :],
            x_v.at[b], sems[b]).wait()

        def load_row(t):
            return [x_v[b, t, pl.ds(j * _LANES, _LANES)] for j in range(nj)]

        def grp_body(gi, _):
            av = attr_v[pl.ds(chunk * _CHUNK + gi * _LANES, _LANES)]
            t0 = gi * _LANES
            row = load_row(t0)
            for k in range(_LANES):
                a = av[k]
                nxt = load_row(t0 + k + 1) if k + 1 < _LANES else None
                for j in range(nj):
                    plsc.addupdate(
                        acc_v.at[a, pl.ds(j * _LANES, _LANES)], row[j])
                row = nxt
            return 0

        lax.fori_loop(0, _CHUNK // _LANES, grp_body, 0)

    issue(0, 0)
    issue(1, 1)

    @pl.loop(0, nsteps - 2, step=2)
    def _(g):
        for b in range(2):
            consume(g + b, b)
            issue(g + b + 2, b)

    for b in range(2):
        consume(nsteps - 2 + b, b)

    pltpu.sync_copy(acc_v, out_hbm.at[wid])


def _sc_segment_sums(attr3, text_feats):
    run = pl.kernel(
        _sc_body,
        out_type=jax.ShapeDtypeStruct((_NTILES, _NSEG, _D), jnp.float32),
        mesh=plsc.VectorSubcoreMesh(core_axis_name="c", subcore_axis_name="s"),
        scratch_types=[
            pltpu.VMEM((_TOK_PER_TILE,), jnp.int32),
            pltpu.VMEM((2, _CHUNK, _D), jnp.float32),
            pltpu.VMEM((_NSEG, _D), jnp.float32),
            pltpu.SemaphoreType.DMA,
            pltpu.SemaphoreType.DMA,
        ],
    )
    return run(attr3, text_feats)


def _tc_body(attr_ref, x_ref, out_ref, cnt_ref):
    attr = attr_ref[0, 0, :]
    x = x_ref[0]
    seg_ids = lax.broadcasted_iota(jnp.int32, (_NSEG, _T), 0)
    mask = (seg_ids == attr[None, :]).astype(jnp.float32)
    out_ref[0] = jnp.dot(mask[:, :_TC_TOK], x,
                         preferred_element_type=jnp.float32)
    cnt_ref[0] = jnp.broadcast_to(
        jnp.sum(mask, axis=1, keepdims=True), (_NSEG, 128))


def _tc_segment_sums(attr3, text_feats):
    return pl.pallas_call(
        _tc_body,
        grid=(_B,),
        in_specs=[
            pl.BlockSpec((1, 1, _T), lambda b: (b, 0, 0)),
            pl.BlockSpec((1, _TC_TOK, _D), lambda b: (b, 0, 0)),
        ],
        out_specs=[
            pl.BlockSpec((1, _NSEG, _D), lambda b: (b, 0, 0)),
            pl.BlockSpec((1, _NSEG, 128), lambda b: (b, 0, 0)),
        ],
        out_shape=[
            jax.ShapeDtypeStruct((_B, _NSEG, _D), jnp.float32),
            jax.ShapeDtypeStruct((_B, _NSEG, 128), jnp.float32),
        ],
    )(attr3, text_feats)


def _epilogue_body(cnt_ref, tc_ref, sc_ref, vg_ref, out_ref):
    counts = cnt_ref[:, :, 0]
    tc = tc_ref[...]
    sc = sc_ref[...]
    vgs = vg_ref[...]

    seg_sums = tc + sc[:, 0] + sc[:, 1]

    mean = seg_sums / counts[:, :, None]
    num = jnp.sum(mean * vgs[:, None, :], axis=2)
    norm_m = jnp.sqrt(jnp.sum(mean * mean, axis=2))
    norm_vg = jnp.sqrt(jnp.sum(vgs * vgs, axis=1, keepdims=True))
    denom = jnp.maximum(norm_vg, _EPS) * jnp.maximum(norm_m, _EPS)
    cos = num / denom

    ids = lax.broadcasted_iota(jnp.int32, (_B, _NSEG), 1)
    present = counts > 0.0
    max_attr = jnp.max(jnp.where(present, ids, 0), axis=1, keepdims=True)
    valid = (ids >= 1) & (ids <= max_attr)
    cs = (jnp.sum(jnp.where(valid, cos, 0.0), axis=1, keepdims=True)
          / max_attr.astype(jnp.float32))
    has_any = max_attr > 0
    loss_b = jnp.where(has_any, 1.0 - cs, 0.0)
    total = jnp.sum(loss_b)
    cnt = jnp.sum(has_any.astype(jnp.float32))
    out_ref[0, 0] = total / cnt


def kernel(attributes, text_feats, Vgs):
    B, T = attributes.shape
    attr3 = attributes.astype(jnp.int32).reshape(B, 1, T)
    sc_part = _sc_segment_sums(attr3, text_feats)
    tc_sums, tc_cnt = _tc_segment_sums(attr3, text_feats)
    out = pl.pallas_call(
        _epilogue_body,
        in_specs=[
            pl.BlockSpec(memory_space=pltpu.VMEM),
            pl.BlockSpec(memory_space=pltpu.VMEM),
            pl.BlockSpec(memory_space=pltpu.VMEM),
            pl.BlockSpec(memory_space=pltpu.VMEM),
        ],
        out_specs=pl.BlockSpec(memory_space=pltpu.SMEM),
        out_shape=jax.ShapeDtypeStruct((1, 1), jnp.float32),
    )(tc_cnt, tc_sums, sc_part.reshape(B, 2, _NSEG, _D), Vgs)
    return out[0, 0]
